# baseline (speedup 1.0000x reference)
import functools

import jax
import jax.numpy as jnp
from jax.experimental import pallas as pl
from jax.experimental.pallas import tpu as pltpu
from jax.sharding import Mesh, PartitionSpec as P


def _fused_body(x_ref, w_ref, b_ref, rw_ref, rb_ref, o_ref):
    m = pl.program_id(1)
    xb = x_ref[...]
    logits = jnp.dot(xb, w_ref[0], preferred_element_type=jnp.float32)
    fusew = jnp.dot(xb, rw_ref[0], preferred_element_type=jnp.float32)
    term = (logits + b_ref[0]) * (fusew + rb_ref[0])

    @pl.when(m == 0)
    def _init():
        o_ref[...] = term

    @pl.when(m != 0)
    def _acc():
        o_ref[...] += term


def _fused_call(xc, mw, mb, rw, rb):
    B, D = xc.shape
    M, _, C = mw.shape
    bB = min(B, 1024)
    grid = (B // bB, M)
    return pl.pallas_call(
        _fused_body,
        grid=grid,
        in_specs=[
            pl.BlockSpec((bB, D), lambda b, m: (b, 0)),
            pl.BlockSpec((1, D, C), lambda b, m: (m, 0, 0)),
            pl.BlockSpec((1, 1, C), lambda b, m: (m, 0, 0)),
            pl.BlockSpec((1, D, C), lambda b, m: (m, 0, 0)),
            pl.BlockSpec((1, 1, C), lambda b, m: (m, 0, 0)),
        ],
        out_specs=pl.BlockSpec((bB, C), lambda b, m: (b, 0)),
        out_shape=jax.ShapeDtypeStruct((B, C), jnp.float32),
        compiler_params=pltpu.CompilerParams(
            dimension_semantics=("parallel", "arbitrary"),
            vmem_limit_bytes=56 * 1024 * 1024,
        ),
    )(xc, mw, mb, rw, rb)


@functools.partial(jax.jit, static_argnames=())
def kernel(x, model_weights, model_bias, resnet_weight, resnet_bias):
    B, D = x.shape
    M, _, C = model_weights.shape

    xc = x.astype(jnp.bfloat16)
    mw = model_weights.astype(jnp.bfloat16)
    rw = resnet_weight.astype(jnp.bfloat16).reshape(D, M, C).transpose(1, 0, 2)
    mb = model_bias.reshape(M, 1, C)
    rb = resnet_bias.reshape(M, 1, C)

    devs = jax.devices()
    n_cores = 2 if (len(devs) >= 2 and B % 2048 == 0) else 1
    if n_cores == 1:
        return _fused_call(xc, mw, mb, rw, rb)

    mesh = Mesh(tuple(devs[:n_cores]), ("b",))
    sharded = jax.shard_map(
        _fused_call,
        mesh=mesh,
        in_specs=(P("b", None), P(), P(), P(), P()),
        out_specs=P("b", None),
        check_vma=False,
    )
    return sharded(xc, mw, mb, rw, rb)


# speedup vs baseline: 1.2681x; 1.2681x over previous
import functools

import jax
import jax.numpy as jnp
from jax.experimental import pallas as pl
from jax.experimental.pallas import tpu as pltpu
from jax.sharding import Mesh, PartitionSpec as P


def _fused_body(x_ref, w_ref, b_ref, rw_ref, rb_ref, o_ref):
    m = pl.program_id(1)
    xb = x_ref[...]
    logits = jnp.dot(xb, w_ref[0], preferred_element_type=jnp.float32)
    fusew = jnp.dot(xb, rw_ref[0], preferred_element_type=jnp.float32)
    term = (logits + b_ref[0]) * (fusew + rb_ref[0])

    @pl.when(m == 0)
    def _init():
        o_ref[...] = term

    @pl.when(m != 0)
    def _acc():
        o_ref[...] += term


def _fused_call(xc, mw, mb, rw, rb):
    B, D = xc.shape
    M, _, C = mw.shape
    bB = min(B, 1024)
    grid = (B // bB, M)
    return pl.pallas_call(
        _fused_body,
        grid=grid,
        in_specs=[
            pl.BlockSpec((bB, D), lambda b, m: (b, 0)),
            pl.BlockSpec((1, D, C), lambda b, m: (m, 0, 0)),
            pl.BlockSpec((1, 1, C), lambda b, m: (m, 0, 0)),
            pl.BlockSpec((1, D, C), lambda b, m: (m, 0, 0)),
            pl.BlockSpec((1, 1, C), lambda b, m: (m, 0, 0)),
        ],
        out_specs=pl.BlockSpec((bB, C), lambda b, m: (b, 0)),
        out_shape=jax.ShapeDtypeStruct((B, C), jnp.float32),
        compiler_params=pltpu.CompilerParams(
            dimension_semantics=("parallel", "arbitrary"),
            vmem_limit_bytes=56 * 1024 * 1024,
        ),
    )(xc, mw, mb, rw, rb)


@functools.partial(jax.jit, static_argnames=())
def kernel(x, model_weights, model_bias, resnet_weight, resnet_bias):
    B, D = x.shape
    M, _, C = model_weights.shape

    rw = resnet_weight.astype(jnp.bfloat16).reshape(D, M, C).transpose(1, 0, 2)
    mb = model_bias.reshape(M, 1, C)
    rb = resnet_bias.reshape(M, 1, C)

    return _fused_call(x, model_weights, mb, rw, rb)
